# Initial kernel scaffold; baseline (speedup 1.0000x reference)
import jax
import jax.numpy as jnp
from jax.experimental import pallas as pl


def kernel(fmap1, fmap2, coords):
    raise NotImplementedError("write your pallas kernel here")


# speedup vs baseline: 14.3910x; 14.3910x over previous
import functools

import numpy as np
import jax
import jax.numpy as jnp
from jax.experimental import pallas as pl
from jax.experimental.pallas import tpu as pltpu

NUM_LEVELS = 4
RADIUS = 3
C = 96
BLK = 64


def _resize_weights(in_size: int, out_size: int) -> np.ndarray:
    scale = out_size / in_size
    inv_scale = 1.0 / scale
    kernel_scale = max(inv_scale, 1.0)
    sample_f = (np.arange(out_size) + 0.5) * inv_scale - 0.5
    x = np.abs(sample_f[:, None] - np.arange(in_size)[None, :]) / kernel_scale
    w = np.maximum(0.0, 1.0 - x)
    total = w.sum(axis=1, keepdims=True)
    w = np.where(np.abs(total) > 1e-8, w / total, 0.0)
    return w.astype(np.float32)


_D = {s: _resize_weights(s, s // 2) for s in (64, 32, 16)}


def _norm_body(x_ref, o_ref):
    x = x_ref[0]
    s = jnp.sum(x * x, axis=-1, keepdims=True)
    o_ref[0] = x / (jnp.sqrt(s) + 1e-6)


def _pad_store(out_ref, data, H, W):
    out_ref[0, pl.ds(3, H), pl.ds(3 * C, W * C)] = data
    col0 = data[:, 0:C]
    for t in range(3):
        out_ref[0, pl.ds(3, H), pl.ds(t * C, C)] = col0
    out_ref[0, pl.ds(3, H), pl.ds((3 + W) * C, 5 * C)] = jnp.zeros(
        (H, 5 * C), jnp.float32)
    row3 = out_ref[0, pl.ds(3, 1), :]
    for t in range(3):
        out_ref[0, pl.ds(t, 1), :] = row3
    out_ref[0, pl.ds(H + 3, 5), :] = jnp.zeros((5, (W + 8) * C), jnp.float32)


def _downsample_w(y, W):
    D = _D[W]
    cols = []
    for p in range(W // 2):
        acc = None
        for w in range(W):
            wgt = float(D[p, w])
            if wgt == 0.0:
                continue
            term = wgt * y[:, w * C:(w + 1) * C]
            acc = term if acc is None else acc + term
        cols.append(acc)
    return jnp.concatenate(cols, axis=-1)


def _pyramid_body(a_ref, d64_ref, d32_ref, d16_ref, p0_ref, p1_ref, p2_ref,
                  p3_ref):
    a0 = a_ref[0]
    _pad_store(p0_ref, a0, 64, 64)
    y1 = jnp.dot(d64_ref[...], a0, preferred_element_type=jnp.float32)
    l1 = _downsample_w(y1, 64)
    _pad_store(p1_ref, l1, 32, 32)
    y2 = jnp.dot(d32_ref[...], l1, preferred_element_type=jnp.float32)
    l2 = _downsample_w(y2, 32)
    _pad_store(p2_ref, l2, 16, 16)
    y3 = jnp.dot(d16_ref[...], l2, preferred_element_type=jnp.float32)
    l3 = _downsample_w(y3, 16)
    _pad_store(p3_ref, l3, 8, 8)


def _sample_body(p0_ref, p1_ref, p2_ref, p3_ref, xv_ref, yv_ref, xs_ref,
                 ys_ref, out_ref, win, wxa, wxb, wya, wyb):
    prefs = (p0_ref, p1_ref, p2_ref, p3_ref)
    for lvl in range(NUM_LEVELS):
        pref = prefs[lvl]
        W = 64 >> lvl
        inv = 1.0 / (1 << lvl)
        xs = xv_ref[0] * inv
        ys = yv_ref[0] * inv
        x0 = xs.astype(jnp.int32)
        y0 = ys.astype(jnp.int32)
        fx = xs - x0.astype(jnp.float32)
        fy = ys - y0.astype(jnp.float32)
        dxs = jax.lax.broadcasted_iota(jnp.int32, (BLK, 7 * C), 1) // C
        mx = (x0 + dxs) <= (W + 1)
        wxa[...] = jnp.where(mx, 1.0 - fx, 0.0)
        wxb[...] = jnp.where(mx, fx, 0.0)
        dys = jax.lax.broadcasted_iota(jnp.int32, (BLK, 8), 1)
        my = (y0 + dys) <= (W + 1)
        wya[...] = jnp.where(my, 1.0 - fy, 0.0)
        wyb[...] = jnp.where(my, fy, 0.0)

        def group(g, _, pref=pref, inv=inv, lvl=lvl):
            base = g * 8
            for p in range(8):
                c0 = (xs_ref[0, 0, 0, base + p] * inv).astype(jnp.int32)
                r0 = (ys_ref[0, 0, 0, base + p] * inv).astype(jnp.int32)
                r_al = pl.multiple_of((r0 // 8) * 8, 8)
                l_al = pl.multiple_of(((c0 * C) // 128) * 128, 128)
                off_r = r0 - r_al
                off_l = c0 * C - l_al
                slab = pref[0, pl.ds(r_al, 16), pl.ds(l_al, 896)]
                slab = pltpu.roll(slab, (16 - off_r) % 16, 0)
                slab = pltpu.roll(slab, (896 - off_l) % 896, 1)
                win[:, p, :] = slab[0:8, 0:8 * C]
            wxa8 = wxa[pl.ds(base, 8), :]
            wxb8 = wxb[pl.ds(base, 8), :]
            wya8 = wya[pl.ds(base, 8), :]
            wyb8 = wyb[pl.ds(base, 8), :]
            rx = []
            for j in range(8):
                wj = win[j]
                rx.append(wxa8 * wj[:, 0:7 * C] + wxb8 * wj[:, C:8 * C])
            for dy in range(7):
                o = wya8[:, dy:dy + 1] * rx[dy] + wyb8[:, dy:dy + 1] * rx[dy + 1]
                out_ref[0, pl.ds(base, 8), lvl * 7 + dy, :] = o
            return 0

        jax.lax.fori_loop(0, BLK // 8, group, 0, unroll=False)


@jax.jit
def kernel(fmap1, fmap2, coords):
    del fmap1
    B, H, W, _ = fmap2.shape
    N = coords.shape[1]

    norm2 = pl.pallas_call(
        _norm_body,
        grid=(B,),
        in_specs=[pl.BlockSpec((1, H, W, C), lambda b: (b, 0, 0, 0))],
        out_specs=pl.BlockSpec((1, H, W, C), lambda b: (b, 0, 0, 0)),
        out_shape=jax.ShapeDtypeStruct((B, H, W, C), jnp.float32),
    )(fmap2)

    a0 = norm2.reshape(B, H, W * C)
    pad_shapes = [(s + 8, (s + 8) * C) for s in (64, 32, 16, 8)]
    pyramid = pl.pallas_call(
        _pyramid_body,
        grid=(B,),
        in_specs=[
            pl.BlockSpec((1, H, W * C), lambda b: (b, 0, 0)),
            pl.BlockSpec((32, 64), lambda b: (0, 0)),
            pl.BlockSpec((16, 32), lambda b: (0, 0)),
            pl.BlockSpec((8, 16), lambda b: (0, 0)),
        ],
        out_specs=[
            pl.BlockSpec((1,) + ps, lambda b: (b, 0, 0)) for ps in pad_shapes
        ],
        out_shape=[
            jax.ShapeDtypeStruct((B,) + ps, jnp.float32) for ps in pad_shapes
        ],
    )(a0, jnp.asarray(_D[64]), jnp.asarray(_D[32]), jnp.asarray(_D[16]))

    xs = coords[..., 0:1]
    ys = coords[..., 1:2]
    xs2 = coords[..., 0].reshape(B, N // BLK, 1, BLK)
    ys2 = coords[..., 1].reshape(B, N // BLK, 1, BLK)
    out = pl.pallas_call(
        _sample_body,
        grid=(B, N // BLK),
        in_specs=[
            pl.BlockSpec((1,) + ps, lambda b, n: (b, 0, 0)) for ps in pad_shapes
        ] + [
            pl.BlockSpec((1, BLK, 1), lambda b, n: (b, n, 0)),
            pl.BlockSpec((1, BLK, 1), lambda b, n: (b, n, 0)),
            pl.BlockSpec((1, 1, 1, BLK), lambda b, n: (b, n, 0, 0),
                         memory_space=pltpu.SMEM),
            pl.BlockSpec((1, 1, 1, BLK), lambda b, n: (b, n, 0, 0),
                         memory_space=pltpu.SMEM),
        ],
        out_specs=pl.BlockSpec((1, BLK, 28, 7 * C), lambda b, n: (b, n, 0, 0)),
        out_shape=jax.ShapeDtypeStruct((B, N, 28, 7 * C), jnp.float32),
        scratch_shapes=[
            pltpu.VMEM((8, 8, 8 * C), jnp.float32),
            pltpu.VMEM((BLK, 7 * C), jnp.float32),
            pltpu.VMEM((BLK, 7 * C), jnp.float32),
            pltpu.VMEM((BLK, 8), jnp.float32),
            pltpu.VMEM((BLK, 8), jnp.float32),
        ],
    )(*pyramid, xs, ys, xs2, ys2)
    return out.reshape(B, N, NUM_LEVELS * 49 * C)
